# baseline (speedup 1.0000x reference)
import functools

import jax
import jax.numpy as jnp
from jax import lax
from jax.experimental import pallas as pl
from jax.experimental.pallas import tpu as pltpu
from jax.experimental.pallas import tpu_sc as plsc

T = 2048
D = 1024
F = 2048
E = 8
K = 2
N = T * K
TM = 128
NT = 40
NP = NT * TM
LANES = 128


def _routing(x2d, gate_w, gate_b):
    logits = x2d @ gate_w + gate_b
    w, se = jax.lax.top_k(logits, K)
    w = jax.nn.softmax(w.astype(jnp.float32), axis=-1)
    return se[:, 0], se[:, 1], w[:, 0], w[:, 1]


def _metadata(e1, e2, w1, w2):
    flat_e = jnp.stack([e1, e2], axis=1).reshape(N)
    flat_w = jnp.stack([w1, w2], axis=1).reshape(N)
    oh = (flat_e[:, None] == jnp.arange(E)[None, :]).astype(jnp.int32)
    cum = jnp.cumsum(oh, axis=0)
    counts = cum[-1]
    rank = jnp.sum((cum - 1) * oh, axis=1)
    tiles_e = (counts + TM - 1) // TM
    bounds = jnp.cumsum(tiles_e)
    row_start = jnp.concatenate([jnp.zeros(1, bounds.dtype), bounds[:-1]]) * TM
    spp = (row_start[flat_e] + rank).astype(jnp.int32)
    spp2 = spp.reshape(T, K)
    p0 = spp2[:, 0]
    p1 = spp2[:, 1]
    spp4 = spp2.reshape(NW, XNC, SCH, K).transpose(0, 1, 3, 2)
    spp4 = spp4.reshape(NW, XNC * K, SCH)
    i = jnp.arange(NT)
    group_id = jnp.sum((bounds[None, :] <= i[:, None]).astype(jnp.int32),
                       axis=1)
    group_id = jnp.minimum(group_id, E - 1).astype(jnp.int32)
    valid = (i < bounds[-1]).astype(jnp.int32)
    gprev = jnp.concatenate([jnp.full((1,), -1, group_id.dtype), group_id[:-1]])
    isfirst = ((group_id != gprev) & (valid == 1)).astype(jnp.int32)
    bufsel = ((jnp.cumsum(isfirst) - 1) % 2).astype(jnp.int32)
    present = counts > 0
    vals = []
    run = jnp.full((), -1, jnp.int32)
    for e in reversed(range(E)):
        vals.append(run)
        run = jnp.where(present[e], jnp.int32(e), run)
    nexte = jnp.stack(vals[::-1])
    nseg = nexte[group_id]
    hasnext = (nseg >= 0).astype(jnp.int32)
    nseg = jnp.maximum(nseg, 0).astype(jnp.int32)
    return spp4, p0, p1, group_id, valid, isfirst, bufsel, nseg, hasnext


_SC_MESH = plsc.VectorSubcoreMesh(core_axis_name="c", subcore_axis_name="s")
NW = 32
SCH = 32
XNC = T // NW // SCH


def _sc_scatter_x(x2d, spp4):
    @functools.partial(
        pl.kernel,
        out_type=jax.ShapeDtypeStruct((NP, D), jnp.float32),
        mesh=_SC_MESH,
        scratch_types=[
            pltpu.VMEM((XNC * K, SCH), jnp.int32),
            pltpu.VMEM((SCH, D), jnp.float32),
            pltpu.VMEM((SCH, D), jnp.float32),
            pltpu.SemaphoreType.DMA,
            pltpu.SemaphoreType.DMA,
            pltpu.SemaphoreType.DMA,
            pltpu.SemaphoreType.DMA,
        ],
    )
    def k(x_hbm, spp_hbm, out_hbm, idx_v, rows0, rows1, sl0, sl1, se, so):
        wid = lax.axis_index("s") * 2 + lax.axis_index("c")
        tbase = wid * (T // NW)
        pltpu.sync_copy(spp_hbm.at[wid], idx_v)
        bufs = (rows0, rows1)
        lsems = (sl0, sl1)
        dl = [None, None]
        dl[0] = pltpu.async_copy(
            x_hbm.at[pl.ds(tbase, SCH)], bufs[0], lsems[0])
        for c in range(XNC):
            if c + 1 < XNC:
                b = (c + 1) % 2
                dl[b] = pltpu.async_copy(
                    x_hbm.at[pl.ds(tbase + (c + 1) * SCH, SCH)],
                    bufs[b], lsems[b])
            cb = c % 2
            dl[cb].wait()
            de = pltpu.async_copy(
                bufs[cb], out_hbm.at[idx_v.at[K * c]], se)
            do = pltpu.async_copy(
                bufs[cb], out_hbm.at[idx_v.at[K * c + 1]], so)
            de.wait()
            do.wait()

    return k(x2d, spp4)


TPW = T // NW
CCH = 16
CNC = TPW // CCH


def _sc_combine(ys, p0, p1, w0, w1):
    @functools.partial(
        pl.kernel,
        out_type=jax.ShapeDtypeStruct((T, D), jnp.float32),
        mesh=_SC_MESH,
        scratch_types=[
            pltpu.VMEM((TPW,), jnp.int32),
            pltpu.VMEM((TPW,), jnp.int32),
            pltpu.VMEM((TPW, 16), jnp.float32),
            pltpu.VMEM((TPW, 16), jnp.float32),
            pltpu.VMEM((CCH, D), jnp.float32),
            pltpu.VMEM((CCH, D), jnp.float32),
            pltpu.VMEM((CCH, D), jnp.float32),
            pltpu.VMEM((CCH, D), jnp.float32),
            pltpu.SemaphoreType.DMA,
            pltpu.SemaphoreType.DMA,
            pltpu.SemaphoreType.DMA,
            pltpu.SemaphoreType.DMA,
        ],
    )
    def k(ys_hbm, p0_hbm, p1_hbm, w0_hbm, w1_hbm, out_hbm, i0_v, i1_v,
          w0_v, w1_v, a0, a1, b0, b1, sa0, sa1, sb0, sb1):
        wid = lax.axis_index("s") * 2 + lax.axis_index("c")
        base = wid * TPW
        lanes_per_row = D // 16
        pltpu.sync_copy(p0_hbm.at[pl.ds(base, TPW)], i0_v)
        pltpu.sync_copy(p1_hbm.at[pl.ds(base, TPW)], i1_v)
        pltpu.sync_copy(w0_hbm.at[pl.ds(base, TPW)], w0_v)
        pltpu.sync_copy(w1_hbm.at[pl.ds(base, TPW)], w1_v)
        abufs = (a0, a1)
        bbufs = (b0, b1)
        asems = (sa0, sa1)
        bsems = (sb0, sb1)
        da = [None, None]
        db = [None, None]
        da[0] = pltpu.async_copy(ys_hbm.at[i0_v.at[pl.ds(0, CCH)]], a0, sa0)
        db[0] = pltpu.async_copy(ys_hbm.at[i1_v.at[pl.ds(0, CCH)]], b0, sb0)
        for c in range(CNC):
            if c + 1 < CNC:
                nb = (c + 1) % 2
                sl_n = pl.ds((c + 1) * CCH, CCH)
                da[nb] = pltpu.async_copy(
                    ys_hbm.at[i0_v.at[sl_n]], abufs[nb], asems[nb])
                db[nb] = pltpu.async_copy(
                    ys_hbm.at[i1_v.at[sl_n]], bbufs[nb], bsems[nb])
            cb = c % 2
            da[cb].wait()
            db[cb].wait()
            av = abufs[cb]
            bv = bbufs[cb]
            roff = c * CCH

            for r in range(CCH):
                w0b = w0_v[roff + r, :]
                w1b = w1_v[roff + r, :]

                def _cc(cc, _2, av=av, bv=bv, r=r, w0b=w0b, w1b=w1b):
                    sl = pl.ds(cc * 16, 16)
                    av[r, sl] = av[r, sl] * w0b + bv[r, sl] * w1b
                    return 0

                lax.fori_loop(0, lanes_per_row, _cc, 0, unroll=8)

            pltpu.sync_copy(av, out_hbm.at[pl.ds(base + c * CCH, CCH)])

    w0x = jnp.broadcast_to(w0[:, None], (T, 16))
    w1x = jnp.broadcast_to(w1[:, None], (T, 16))
    return k(ys, p0, p1, w0x, w1x)


def _ffn_body(gid_ref, valid_ref, isf_ref, bsel_ref, nseg_ref, hn_ref,
              xs_ref, w1_hbm, b1_ref, w2_hbm, b2_ref, ys_ref,
              w1b, w2b, sems):
    i = pl.program_id(0)

    def w1_dma(e, b):
        return pltpu.make_async_copy(w1_hbm.at[e], w1b.at[b], sems.at[b])

    def w2_dma(e, b):
        return pltpu.make_async_copy(w2_hbm.at[e], w2b.at[b], sems.at[2 + b])

    @pl.when(i == 0)
    def _():
        e0 = gid_ref[0]
        w1_dma(e0, 0).start()
        w2_dma(e0, 0).start()

        @pl.when(hn_ref[0] == 1)
        def _():
            e1 = nseg_ref[0]
            w1_dma(e1, 1).start()
            w2_dma(e1, 1).start()

    @pl.when(isf_ref[i] == 1)
    def _():
        b = bsel_ref[i]
        w1_dma(gid_ref[i], b).wait()
        w2_dma(gid_ref[i], b).wait()

        @pl.when((i > 0) & (hn_ref[i] == 1))
        def _():
            e_n = nseg_ref[i]
            w1_dma(e_n, 1 - b).start()
            w2_dma(e_n, 1 - b).start()

    def compute(b):
        h = jnp.dot(xs_ref[...], w1b[b],
                    preferred_element_type=jnp.float32) + b1_ref[0]
        h = jnp.maximum(h, 0.0)
        ys_ref[...] = jnp.dot(h, w2b[b],
                              preferred_element_type=jnp.float32) + b2_ref[0]

    @pl.when((valid_ref[i] == 1) & (bsel_ref[i] == 0))
    def _():
        compute(0)

    @pl.when((valid_ref[i] == 1) & (bsel_ref[i] == 1))
    def _():
        compute(1)


def _grouped_ffn(xs, W1, b1, W2, b2, group_id, valid, isfirst, bufsel,
                 nseg, hasnext):
    grid_spec = pltpu.PrefetchScalarGridSpec(
        num_scalar_prefetch=6,
        grid=(NT,),
        in_specs=[
            pl.BlockSpec((TM, D), lambda i, *_: (i, 0)),
            pl.BlockSpec(memory_space=pl.ANY),
            pl.BlockSpec((1, 1, F), lambda i, g, *_: (g[i], 0, 0)),
            pl.BlockSpec(memory_space=pl.ANY),
            pl.BlockSpec((1, 1, D), lambda i, g, *_: (g[i], 0, 0)),
        ],
        out_specs=pl.BlockSpec((TM, D), lambda i, *_: (i, 0)),
        scratch_shapes=[
            pltpu.VMEM((2, D, F), jnp.float32),
            pltpu.VMEM((2, F, D), jnp.float32),
            pltpu.SemaphoreType.DMA((4,)),
        ],
    )
    return pl.pallas_call(
        _ffn_body,
        grid_spec=grid_spec,
        out_shape=jax.ShapeDtypeStruct((NP, D), jnp.float32),
    )(group_id, valid, isfirst, bufsel, nseg, hasnext,
      xs, W1, b1.reshape(E, 1, F), W2, b2.reshape(E, 1, D))


def kernel(inputs, gate_w, gate_b, W1, b1, W2, b2):
    x2d = inputs.reshape(T, D)
    e1, e2, w1c, w2c = _routing(x2d, gate_w, gate_b)
    (spp4, p0, p1, group_id, valid, isfirst, bufsel, nseg,
     hasnext) = _metadata(e1, e2, w1c, w2c)
    xs = _sc_scatter_x(x2d, spp4)
    ys = _grouped_ffn(xs, W1, b1, W2, b2, group_id, valid, isfirst, bufsel,
                      nseg, hasnext)
    out = _sc_combine(ys, p0, p1, w1c, w2c)
    return out.reshape(inputs.shape)


# speedup vs baseline: 1.0349x; 1.0349x over previous
import functools

import jax
import jax.numpy as jnp
from jax import lax
from jax.experimental import pallas as pl
from jax.experimental.pallas import tpu as pltpu
from jax.experimental.pallas import tpu_sc as plsc

T = 2048
D = 1024
F = 2048
E = 8
K = 2
N = T * K
TM = 128
NT = 40
NP = NT * TM


def _routing(x2d, gate_w, gate_b):
    logits = x2d @ gate_w + gate_b
    w, se = jax.lax.top_k(logits, K)
    w = jax.nn.softmax(w.astype(jnp.float32), axis=-1)
    return se[:, 0], se[:, 1], w[:, 0], w[:, 1]


def _metadata(e1, e2):
    flat_e = jnp.stack([e1, e2], axis=1).reshape(N)
    oh = (flat_e[:, None] == jnp.arange(E)[None, :]).astype(jnp.int32)
    cum = jnp.cumsum(oh, axis=0)
    counts = cum[-1]
    rank = jnp.sum((cum - 1) * oh, axis=1)
    tiles_e = (counts + TM - 1) // TM
    bounds = jnp.cumsum(tiles_e)
    row_start = jnp.concatenate([jnp.zeros(1, bounds.dtype), bounds[:-1]]) * TM
    spp = (row_start[flat_e] + rank).astype(jnp.int32)
    spp2 = spp.reshape(T, K)
    p0 = spp2[:, 0]
    p1 = spp2[:, 1]
    spp4 = spp2.reshape(NW, XNC, SCH, K).transpose(0, 1, 3, 2)
    spp4 = spp4.reshape(NW, XNC * K, SCH)
    i = jnp.arange(NT)
    group_id = jnp.sum((bounds[None, :] <= i[:, None]).astype(jnp.int32),
                       axis=1)
    group_id = jnp.minimum(group_id, E - 1).astype(jnp.int32)
    valid = (i < bounds[-1]).astype(jnp.int32)
    gprev = jnp.concatenate([jnp.full((1,), -1, group_id.dtype), group_id[:-1]])
    isfirst = ((group_id != gprev) & (valid == 1)).astype(jnp.int32)
    bufsel = ((jnp.cumsum(isfirst) - 1) % 2).astype(jnp.int32)
    present = counts > 0
    vals = []
    run = jnp.full((), -1, jnp.int32)
    for e in reversed(range(E)):
        vals.append(run)
        run = jnp.where(present[e], jnp.int32(e), run)
    nexte = jnp.stack(vals[::-1])
    nseg = nexte[group_id]
    hasnext = (nseg >= 0).astype(jnp.int32)
    nseg = jnp.maximum(nseg, 0).astype(jnp.int32)
    return spp4, p0, p1, group_id, valid, isfirst, bufsel, nseg, hasnext


_SC_MESH = plsc.VectorSubcoreMesh(core_axis_name="c", subcore_axis_name="s")
NW = 32
SCH = 32
XNC = T // NW // SCH


def _sc_scatter_x(x2d, spp4):
    @functools.partial(
        pl.kernel,
        out_type=jax.ShapeDtypeStruct((NP, D), jnp.float32),
        mesh=_SC_MESH,
        scratch_types=[
            pltpu.VMEM((XNC * K, SCH), jnp.int32),
            pltpu.VMEM((SCH, D), jnp.float32),
            pltpu.VMEM((SCH, D), jnp.float32),
            pltpu.SemaphoreType.DMA,
            pltpu.SemaphoreType.DMA,
            pltpu.SemaphoreType.DMA,
            pltpu.SemaphoreType.DMA,
        ],
    )
    def k(x_hbm, spp_hbm, out_hbm, idx_v, rows0, rows1, sl0, sl1, se, so):
        wid = lax.axis_index("s") * 2 + lax.axis_index("c")
        tbase = wid * (T // NW)
        pltpu.sync_copy(spp_hbm.at[wid], idx_v)
        bufs = (rows0, rows1)
        lsems = (sl0, sl1)
        dl = [None, None]
        dl[0] = pltpu.async_copy(
            x_hbm.at[pl.ds(tbase, SCH)], bufs[0], lsems[0])
        for c in range(XNC):
            if c + 1 < XNC:
                b = (c + 1) % 2
                dl[b] = pltpu.async_copy(
                    x_hbm.at[pl.ds(tbase + (c + 1) * SCH, SCH)],
                    bufs[b], lsems[b])
            cb = c % 2
            dl[cb].wait()
            de = pltpu.async_copy(
                bufs[cb], out_hbm.at[idx_v.at[K * c]], se)
            do = pltpu.async_copy(
                bufs[cb], out_hbm.at[idx_v.at[K * c + 1]], so)
            de.wait()
            do.wait()

    return k(x2d, spp4)


TPW = T // NW
CCH = 16
CNC = TPW // CCH


def _sc_combine(ys, p0, p1, w0, w1):
    @functools.partial(
        pl.kernel,
        out_type=jax.ShapeDtypeStruct((T, D), jnp.float32),
        mesh=_SC_MESH,
        scratch_types=[
            pltpu.VMEM((TPW,), jnp.int32),
            pltpu.VMEM((TPW,), jnp.int32),
            pltpu.VMEM((TPW, 16), jnp.float32),
            pltpu.VMEM((TPW, 16), jnp.float32),
            pltpu.VMEM((CCH, D), jnp.float32),
            pltpu.VMEM((CCH, D), jnp.float32),
            pltpu.VMEM((CCH, D), jnp.float32),
            pltpu.VMEM((CCH, D), jnp.float32),
            pltpu.SemaphoreType.DMA,
            pltpu.SemaphoreType.DMA,
            pltpu.SemaphoreType.DMA,
            pltpu.SemaphoreType.DMA,
        ],
    )
    def k(ys_hbm, p0_hbm, p1_hbm, w0_hbm, w1_hbm, out_hbm, i0_v, i1_v,
          w0_v, w1_v, a0, a1, b0, b1, sa0, sa1, sb0, sb1):
        wid = lax.axis_index("s") * 2 + lax.axis_index("c")
        base = wid * TPW
        lanes_per_row = D // 16
        pltpu.sync_copy(p0_hbm.at[pl.ds(base, TPW)], i0_v)
        pltpu.sync_copy(p1_hbm.at[pl.ds(base, TPW)], i1_v)
        pltpu.sync_copy(w0_hbm.at[pl.ds(base, TPW)], w0_v)
        pltpu.sync_copy(w1_hbm.at[pl.ds(base, TPW)], w1_v)
        abufs = (a0, a1)
        bbufs = (b0, b1)
        asems = (sa0, sa1)
        bsems = (sb0, sb1)
        da = [None, None]
        db = [None, None]
        da[0] = pltpu.async_copy(ys_hbm.at[i0_v.at[pl.ds(0, CCH)]], a0, sa0)
        db[0] = pltpu.async_copy(ys_hbm.at[i1_v.at[pl.ds(0, CCH)]], b0, sb0)
        for c in range(CNC):
            if c + 1 < CNC:
                nb = (c + 1) % 2
                sl_n = pl.ds((c + 1) * CCH, CCH)
                da[nb] = pltpu.async_copy(
                    ys_hbm.at[i0_v.at[sl_n]], abufs[nb], asems[nb])
                db[nb] = pltpu.async_copy(
                    ys_hbm.at[i1_v.at[sl_n]], bbufs[nb], bsems[nb])
            cb = c % 2
            da[cb].wait()
            db[cb].wait()
            av = abufs[cb]
            bv = bbufs[cb]
            roff = c * CCH

            def _row(r, _, av=av, bv=bv, roff=roff):
                w0b = w0_v[roff + r, :]
                w1b = w1_v[roff + r, :]

                def _cc(cc, _2, av=av, bv=bv, r=r, w0b=w0b, w1b=w1b):
                    sl = pl.ds(cc * 16, 16)
                    av[r, sl] = av[r, sl] * w0b + bv[r, sl] * w1b
                    return 0

                lax.fori_loop(0, lanes_per_row, _cc, 0, unroll=8)
                return 0

            lax.fori_loop(0, CCH, _row, 0)

            pltpu.sync_copy(av, out_hbm.at[pl.ds(base + c * CCH, CCH)])

    w0x = jnp.broadcast_to(w0[:, None], (T, 16))
    w1x = jnp.broadcast_to(w1[:, None], (T, 16))
    return k(ys, p0, p1, w0x, w1x)


def _ffn_body(gid_ref, valid_ref, isf_ref, bsel_ref, nseg_ref, hn_ref,
              xs_ref, w1_hbm, b1_ref, w2_hbm, b2_ref, ys_ref,
              w1b, w2b, sems):
    i = pl.program_id(0)

    def w1_dma(e, b):
        return pltpu.make_async_copy(w1_hbm.at[e], w1b.at[b], sems.at[b])

    def w2_dma(e, b):
        return pltpu.make_async_copy(w2_hbm.at[e], w2b.at[b], sems.at[2 + b])

    @pl.when(i == 0)
    def _():
        e0 = gid_ref[0]
        w1_dma(e0, 0).start()
        w2_dma(e0, 0).start()

        @pl.when(hn_ref[0] == 1)
        def _():
            e1 = nseg_ref[0]
            w1_dma(e1, 1).start()
            w2_dma(e1, 1).start()

    @pl.when(isf_ref[i] == 1)
    def _():
        b = bsel_ref[i]
        w1_dma(gid_ref[i], b).wait()
        w2_dma(gid_ref[i], b).wait()

        @pl.when((i > 0) & (hn_ref[i] == 1))
        def _():
            e_n = nseg_ref[i]
            w1_dma(e_n, 1 - b).start()
            w2_dma(e_n, 1 - b).start()

    def compute(b):
        h = jnp.dot(xs_ref[...], w1b[b],
                    preferred_element_type=jnp.float32) + b1_ref[0]
        h = jnp.maximum(h, 0.0)
        ys_ref[...] = jnp.dot(h, w2b[b],
                              preferred_element_type=jnp.float32) + b2_ref[0]

    @pl.when((valid_ref[i] == 1) & (bsel_ref[i] == 0))
    def _():
        compute(0)

    @pl.when((valid_ref[i] == 1) & (bsel_ref[i] == 1))
    def _():
        compute(1)


def _grouped_ffn(xs, W1, b1, W2, b2, group_id, valid, isfirst, bufsel,
                 nseg, hasnext):
    grid_spec = pltpu.PrefetchScalarGridSpec(
        num_scalar_prefetch=6,
        grid=(NT,),
        in_specs=[
            pl.BlockSpec((TM, D), lambda i, *_: (i, 0)),
            pl.BlockSpec(memory_space=pl.ANY),
            pl.BlockSpec((1, 1, F), lambda i, g, *_: (g[i], 0, 0)),
            pl.BlockSpec(memory_space=pl.ANY),
            pl.BlockSpec((1, 1, D), lambda i, g, *_: (g[i], 0, 0)),
        ],
        out_specs=pl.BlockSpec((TM, D), lambda i, *_: (i, 0)),
        scratch_shapes=[
            pltpu.VMEM((2, D, F), jnp.float32),
            pltpu.VMEM((2, F, D), jnp.float32),
            pltpu.SemaphoreType.DMA((4,)),
        ],
    )
    return pl.pallas_call(
        _ffn_body,
        grid_spec=grid_spec,
        out_shape=jax.ShapeDtypeStruct((NP, D), jnp.float32),
    )(group_id, valid, isfirst, bufsel, nseg, hasnext,
      xs, W1, b1.reshape(E, 1, F), W2, b2.reshape(E, 1, D))


def kernel(inputs, gate_w, gate_b, W1, b1, W2, b2):
    x2d = inputs.reshape(T, D)
    e1, e2, w1c, w2c = _routing(x2d, gate_w, gate_b)
    (spp4, p0, p1, group_id, valid, isfirst, bufsel, nseg,
     hasnext) = _metadata(e1, e2)
    xs = _sc_scatter_x(x2d, spp4)
    ys = _grouped_ffn(xs, W1, b1, W2, b2, group_id, valid, isfirst, bufsel,
                      nseg, hasnext)
    out = _sc_combine(ys, p0, p1, w1c, w2c)
    return out.reshape(inputs.shape)
